# baseline (speedup 1.0000x reference)
import jax
import jax.numpy as jnp
from jax import lax
from jax.experimental import pallas as pl
from jax.experimental.pallas import tpu as pltpu
from jax.experimental.pallas import tpu_sc as plsc

N_EDGE = 10000
N_LINE = 320000
HIDDEN = 128
ATTN = 64

NC = 2
NS = 16
NW = NC * NS
LINES_PER_TILE = N_LINE // NW
CHUNK = 80
NCHUNK = LINES_PER_TILE // CHUNK
IDXBLK = 25
NBLKS = NCHUNK // IDXBLK
N_PAD = 10240
ROWS_PER_TILE = N_PAD // NS


def _proj_body(eh_ref, qr_ref, ws_ref, wd_ref, wq_ref, bq_ref, wm_ref,
               a_ref, q_ref, m_ref):
    eh = eh_ref[...]
    qr = qr_ref[...]
    a = jnp.dot(eh, ws_ref[...], preferred_element_type=jnp.float32)
    bc = (jnp.dot(eh, wd_ref[...], preferred_element_type=jnp.float32)
          + jnp.dot(qr, wq_ref[...], preferred_element_type=jnp.float32)
          + bq_ref[...])
    a_ref[...] = a
    q_ref[...] = jnp.concatenate([bc, jnp.zeros_like(bc)], axis=1)
    m_ref[...] = jnp.dot(eh, wm_ref[...], preferred_element_type=jnp.float32)


def _out_body(p0_ref, p1_ref, wo_ref, o_ref):
    o_ref[...] = jnp.dot(p0_ref[...] + p1_ref[...], wo_ref[...],
                         preferred_element_type=jnp.float32)


def _sc_body(u_hbm, q_hbm, src4_hbm, dst4_hbm, wa_hbm, ba_hbm,
             out_hbm, agg_sh, sidx_v, didx_v, uv, omsg_v, wa_v, ba_v,
             sem_a, sem_g):
    cid = lax.axis_index("c")
    sid = lax.axis_index("s")
    wid = cid * NS + sid

    def zrow(r, carry):
        for k in range(HIDDEN // 16):
            omsg_v[r, pl.ds(k * 16, 16)] = jnp.zeros((16,), jnp.float32)
        return carry
    lax.fori_loop(0, CHUNK, zrow, 0)
    for j in range(ROWS_PER_TILE // CHUNK):
        pltpu.sync_copy(omsg_v, agg_sh.at[pl.ds(sid * ROWS_PER_TILE + j * CHUNK, CHUNK)])

    pltpu.sync_copy(wa_hbm, wa_v)
    pltpu.sync_copy(ba_hbm, ba_v)
    plsc.subcore_barrier()

    def issue_g1(r, par):
        pltpu.async_copy(u_hbm.at[sidx_v.at[r]], uv.at[par], sem_a)

    def wait_g1(r, par):
        pltpu.make_async_copy(u_hbm.at[sidx_v.at[r]], uv.at[par], sem_a).wait()

    def issue_g2(r, par):
        pltpu.async_copy(q_hbm.at[didx_v.at[r]], uv.at[par], sem_g, add=True)

    def wait_g2(r, par):
        pltpu.make_async_copy(q_hbm.at[didx_v.at[r]], uv.at[par], sem_g).wait()

    def chunk_step(r, carry):
        par = jnp.bitwise_and(r, 1)
        npar = 1 - par
        wait_g2(r, par)

        @pl.when(r < IDXBLK - 1)
        def _():
            wait_g1(r + 1, npar)
            issue_g2(r + 1, npar)

        wa = [wa_v[pl.ds(k * 16, 16)] for k in range(ATTN // 16)]
        bvec = ba_v[...]

        def line_body(g, c2):
            for u in range(8):
                i = g * 8 + u
                p = []
                for k in range(ATTN // 16):
                    pre = uv[par, i, pl.ds(k * 16, 16)]
                    p.append(jnp.maximum(pre, 0.0) * wa[k])
                s = jnp.sum((p[0] + p[1]) + (p[2] + p[3]))
                z = s + bvec
                alpha = 1.0 / (1.0 + jnp.exp(-z))
                for k in range(4):
                    w = uv[par, i, pl.ds(ATTN + k * 16, 16)]
                    lo, hi = plsc.unpack(plsc.bitcast(w, jnp.bfloat16),
                                         format=plsc.PackFormat.INTERLEAVED,
                                         preferred_element_type=jnp.float32)
                    omsg_v[i, pl.ds(k * 32, 16)] = lo * alpha
                    omsg_v[i, pl.ds(k * 32 + 16, 16)] = hi * alpha
            return c2
        lax.fori_loop(0, CHUNK // 8, line_body, 0)

        @pl.when(r < IDXBLK - 2)
        def _():
            issue_g1(r + 2, par)

        pltpu.sync_copy(omsg_v, agg_sh.at[didx_v.at[r]], add=True)
        return carry

    for blk in range(NBLKS):
        pltpu.sync_copy(src4_hbm.at[wid, blk], sidx_v)
        pltpu.sync_copy(dst4_hbm.at[wid, blk], didx_v)
        issue_g1(0, 0)
        wait_g1(0, 0)
        issue_g2(0, 0)
        issue_g1(1, 1)
        lax.fori_loop(0, IDXBLK, chunk_step, 0)

    plsc.subcore_barrier()
    pltpu.sync_copy(agg_sh.at[pl.ds(sid * ROWS_PER_TILE, ROWS_PER_TILE)],
                    out_hbm.at[cid, pl.ds(sid * ROWS_PER_TILE, ROWS_PER_TILE)])


def kernel(edge_hidden, q_rel_emb, line_src, line_dst, n_edge, W_src, W_dst,
           W_qr, b_qr, w_alpha, b_alpha, W_msg, W_out):
    n = edge_hidden.shape[0]
    blk = 1000
    grid = n // blk

    a_proj, q_proj, m_proj = pl.pallas_call(
        _proj_body,
        grid=(grid,),
        in_specs=[
            pl.BlockSpec((blk, HIDDEN), lambda i: (i, 0)),
            pl.BlockSpec((blk, HIDDEN), lambda i: (i, 0)),
            pl.BlockSpec((HIDDEN, ATTN), lambda i: (0, 0)),
            pl.BlockSpec((HIDDEN, ATTN), lambda i: (0, 0)),
            pl.BlockSpec((HIDDEN, ATTN), lambda i: (0, 0)),
            pl.BlockSpec((1, ATTN), lambda i: (0, 0)),
            pl.BlockSpec((HIDDEN, HIDDEN), lambda i: (0, 0)),
        ],
        out_specs=[
            pl.BlockSpec((blk, ATTN), lambda i: (i, 0)),
            pl.BlockSpec((blk, 2 * ATTN), lambda i: (i, 0)),
            pl.BlockSpec((blk, HIDDEN), lambda i: (i, 0)),
        ],
        out_shape=[
            jax.ShapeDtypeStruct((n, ATTN), jnp.float32),
            jax.ShapeDtypeStruct((n, 2 * ATTN), jnp.float32),
            jax.ShapeDtypeStruct((n, HIDDEN), jnp.float32),
        ],
    )(edge_hidden, q_rel_emb, W_src, W_dst, W_qr, b_qr.reshape(1, ATTN), W_msg)

    m16 = m_proj.astype(jnp.bfloat16).reshape(n, 4, 2, 16)
    mpk = jnp.stack([m16[:, :, 0, :], m16[:, :, 1, :]], axis=-1)
    mpk = lax.bitcast_convert_type(mpk, jnp.float32).reshape(n, ATTN)
    u_tab = jnp.concatenate([a_proj, mpk], axis=1)

    wa_flat = w_alpha.reshape(ATTN)
    ba_vec = jnp.broadcast_to(b_alpha.reshape(1), (16,))
    src32 = line_src.astype(jnp.int32).reshape(NW, NBLKS, IDXBLK, CHUNK)
    dst32 = line_dst.astype(jnp.int32).reshape(NW, NBLKS, IDXBLK, CHUNK)

    parts = pl.kernel(
        _sc_body,
        out_type=jax.ShapeDtypeStruct((NC, N_PAD, HIDDEN), jnp.float32),
        mesh=plsc.VectorSubcoreMesh(core_axis_name="c", subcore_axis_name="s",
                                    num_cores=NC, num_subcores=NS),
        compiler_params=pltpu.CompilerParams(needs_layout_passes=False),
        scratch_types=[
            pltpu.VMEM_SHARED((N_PAD, HIDDEN), jnp.float32),
            pltpu.VMEM((IDXBLK, CHUNK), jnp.int32),
            pltpu.VMEM((IDXBLK, CHUNK), jnp.int32),
            pltpu.VMEM((2, CHUNK, HIDDEN), jnp.float32),
            pltpu.VMEM((CHUNK, HIDDEN), jnp.float32),
            pltpu.VMEM((ATTN,), jnp.float32),
            pltpu.VMEM((16,), jnp.float32),
            pltpu.SemaphoreType.DMA,
            pltpu.SemaphoreType.DMA,
        ],
    )(u_tab, q_proj, src32, dst32, wa_flat, ba_vec)

    hidden_new = pl.pallas_call(
        _out_body,
        grid=(grid,),
        in_specs=[
            pl.BlockSpec((blk, HIDDEN), lambda i: (i, 0)),
            pl.BlockSpec((blk, HIDDEN), lambda i: (i, 0)),
            pl.BlockSpec((HIDDEN, HIDDEN), lambda i: (0, 0)),
        ],
        out_specs=pl.BlockSpec((blk, HIDDEN), lambda i: (i, 0)),
        out_shape=jax.ShapeDtypeStruct((n, HIDDEN), jnp.float32),
    )(parts[0], parts[1], W_out)

    return hidden_new + jnp.zeros((), dtype=hidden_new.dtype) * n_edge


# speedup vs baseline: 1.0273x; 1.0273x over previous
import jax
import jax.numpy as jnp
from jax import lax
from jax.experimental import pallas as pl
from jax.experimental.pallas import tpu as pltpu
from jax.experimental.pallas import tpu_sc as plsc

N_EDGE = 10000
N_LINE = 320000
HIDDEN = 128
ATTN = 64

NC = 2
NS = 16
NW = NC * NS
LINES_PER_TILE = N_LINE // NW
CHUNK = 40
NCHUNK = LINES_PER_TILE // CHUNK
IDXBLK = 25
NBLKS = NCHUNK // IDXBLK
N_PAD = 10112
ROWS_PER_TILE = N_PAD // NS


def _proj_body(eh_ref, qr_ref, ws_ref, wd_ref, wq_ref, bq_ref, wm_ref,
               p_ref, q_ref, m_ref):
    eh = eh_ref[...]
    qr = qr_ref[...]
    a = jnp.dot(eh, ws_ref[...], preferred_element_type=jnp.float32)
    bc = (jnp.dot(eh, wd_ref[...], preferred_element_type=jnp.float32)
          + jnp.dot(qr, wq_ref[...], preferred_element_type=jnp.float32)
          + bq_ref[...])
    p_ref[...] = jnp.concatenate([a, bc], axis=1)
    q_ref[...] = jnp.concatenate([bc, a], axis=1)
    m_ref[...] = jnp.dot(eh, wm_ref[...], preferred_element_type=jnp.float32)


def _out_body(p0_ref, p1_ref, wo_ref, o_ref):
    o_ref[...] = jnp.dot(p0_ref[...] + p1_ref[...], wo_ref[...],
                         preferred_element_type=jnp.float32)


def _sc_body(u_hbm, q_hbm, src4_hbm, dst4_hbm, wa_hbm, ba_hbm,
             out_hbm, agg_sh, sidx_v, didx_v, uv, qv, omsg_v, wa_v, ba_v,
             sem_g):
    cid = lax.axis_index("c")
    sid = lax.axis_index("s")
    wid = cid * NS + sid

    def zrow(r, carry):
        for k in range(HIDDEN // 16):
            omsg_v[r, pl.ds(k * 16, 16)] = jnp.zeros((16,), jnp.float32)
        return carry
    lax.fori_loop(0, CHUNK, zrow, 0)
    for j in range(ROWS_PER_TILE // CHUNK):
        pltpu.sync_copy(omsg_v, agg_sh.at[pl.ds(sid * ROWS_PER_TILE + j * CHUNK, CHUNK)])
    pltpu.sync_copy(omsg_v.at[pl.ds(0, ROWS_PER_TILE % CHUNK)],
                    agg_sh.at[pl.ds(sid * ROWS_PER_TILE + (ROWS_PER_TILE // CHUNK) * CHUNK,
                                    ROWS_PER_TILE % CHUNK)])

    pltpu.sync_copy(wa_hbm, wa_v)
    pltpu.sync_copy(ba_hbm, ba_v)
    plsc.subcore_barrier()

    def issue_gathers(r):
        s = lax.rem(r, 3)
        pltpu.async_copy(u_hbm.at[sidx_v.at[r]], uv.at[s], sem_g)
        pltpu.async_copy(q_hbm.at[didx_v.at[r]], qv.at[s], sem_g)

    def wait_gathers(r):
        s = lax.rem(r, 3)
        pltpu.make_async_copy(u_hbm.at[sidx_v.at[r]], uv.at[s], sem_g).wait()
        pltpu.make_async_copy(q_hbm.at[didx_v.at[r]], qv.at[s], sem_g).wait()

    def chunk_step(r, carry):
        sdyn = lax.rem(r, 3)
        wait_gathers(r)

        @pl.when(r < IDXBLK - 2)
        def _():
            issue_gathers(r + 2)

        wa = [wa_v[pl.ds(k * 16, 16)] for k in range(ATTN // 16)]
        bvec = ba_v[...]

        def line_body(g, c2):
            for u in range(8):
                i = g * 8 + u
                p = []
                for k in range(ATTN // 16):
                    pre = uv[sdyn, i, pl.ds(k * 16, 16)] + qv[sdyn, i, pl.ds(k * 16, 16)]
                    p.append(jnp.maximum(pre, 0.0) * wa[k])
                s = jnp.sum((p[0] + p[1]) + (p[2] + p[3]))
                z = s + bvec
                alpha = 1.0 / (1.0 + jnp.exp(-z))
                for k in range(4):
                    w = uv[sdyn, i, pl.ds(ATTN + k * 16, 16)]
                    lo, hi = plsc.unpack(plsc.bitcast(w, jnp.bfloat16),
                                         format=plsc.PackFormat.INTERLEAVED,
                                         preferred_element_type=jnp.float32)
                    omsg_v[i, pl.ds(k * 32, 16)] = lo * alpha
                    omsg_v[i, pl.ds(k * 32 + 16, 16)] = hi * alpha
            return c2
        lax.fori_loop(0, CHUNK // 8, line_body, 0)
        pltpu.sync_copy(omsg_v, agg_sh.at[didx_v.at[r]], add=True)
        return carry

    def block_body(blk, carry):
        pltpu.sync_copy(src4_hbm.at[wid, blk], sidx_v)
        pltpu.sync_copy(dst4_hbm.at[wid, blk], didx_v)
        issue_gathers(0)
        issue_gathers(1)
        lax.fori_loop(0, IDXBLK, chunk_step, 0)
        return carry
    lax.fori_loop(0, NBLKS, block_body, 0)

    plsc.subcore_barrier()
    pltpu.sync_copy(agg_sh.at[pl.ds(sid * ROWS_PER_TILE, ROWS_PER_TILE)],
                    out_hbm.at[cid, pl.ds(sid * ROWS_PER_TILE, ROWS_PER_TILE)])


def kernel(edge_hidden, q_rel_emb, line_src, line_dst, n_edge, W_src, W_dst,
           W_qr, b_qr, w_alpha, b_alpha, W_msg, W_out):
    n = edge_hidden.shape[0]
    blk = 1000
    grid = n // blk

    p_proj, q_proj, m_proj = pl.pallas_call(
        _proj_body,
        grid=(grid,),
        in_specs=[
            pl.BlockSpec((blk, HIDDEN), lambda i: (i, 0)),
            pl.BlockSpec((blk, HIDDEN), lambda i: (i, 0)),
            pl.BlockSpec((HIDDEN, ATTN), lambda i: (0, 0)),
            pl.BlockSpec((HIDDEN, ATTN), lambda i: (0, 0)),
            pl.BlockSpec((HIDDEN, ATTN), lambda i: (0, 0)),
            pl.BlockSpec((1, ATTN), lambda i: (0, 0)),
            pl.BlockSpec((HIDDEN, HIDDEN), lambda i: (0, 0)),
        ],
        out_specs=[
            pl.BlockSpec((blk, 2 * ATTN), lambda i: (i, 0)),
            pl.BlockSpec((blk, 2 * ATTN), lambda i: (i, 0)),
            pl.BlockSpec((blk, HIDDEN), lambda i: (i, 0)),
        ],
        out_shape=[
            jax.ShapeDtypeStruct((n, 2 * ATTN), jnp.float32),
            jax.ShapeDtypeStruct((n, 2 * ATTN), jnp.float32),
            jax.ShapeDtypeStruct((n, HIDDEN), jnp.float32),
        ],
    )(edge_hidden, q_rel_emb, W_src, W_dst, W_qr, b_qr.reshape(1, ATTN), W_msg)

    m16 = m_proj.astype(jnp.bfloat16).reshape(n, 4, 2, 16)
    mpk = jnp.stack([m16[:, :, 0, :], m16[:, :, 1, :]], axis=-1)
    mpk = lax.bitcast_convert_type(mpk, jnp.float32).reshape(n, ATTN)
    u_tab = jnp.concatenate([p_proj[:, :ATTN], mpk], axis=1)

    wa_flat = w_alpha.reshape(ATTN)
    ba_vec = jnp.broadcast_to(b_alpha.reshape(1), (16,))
    src32 = line_src.astype(jnp.int32).reshape(NW, NBLKS, IDXBLK, CHUNK)
    dst32 = line_dst.astype(jnp.int32).reshape(NW, NBLKS, IDXBLK, CHUNK)

    parts = pl.kernel(
        _sc_body,
        out_type=jax.ShapeDtypeStruct((NC, N_PAD, HIDDEN), jnp.float32),
        mesh=plsc.VectorSubcoreMesh(core_axis_name="c", subcore_axis_name="s",
                                    num_cores=NC, num_subcores=NS),
        compiler_params=pltpu.CompilerParams(needs_layout_passes=False),
        scratch_types=[
            pltpu.VMEM_SHARED((N_PAD, HIDDEN), jnp.float32),
            pltpu.VMEM((IDXBLK, CHUNK), jnp.int32),
            pltpu.VMEM((IDXBLK, CHUNK), jnp.int32),
            pltpu.VMEM((3, CHUNK, HIDDEN), jnp.float32),
            pltpu.VMEM((3, CHUNK, HIDDEN), jnp.float32),
            pltpu.VMEM((CHUNK, HIDDEN), jnp.float32),
            pltpu.VMEM((ATTN,), jnp.float32),
            pltpu.VMEM((16,), jnp.float32),
            pltpu.SemaphoreType.DMA,
        ],
    )(u_tab, q_proj, src32, dst32, wa_flat, ba_vec)

    hidden_new = pl.pallas_call(
        _out_body,
        grid=(grid,),
        in_specs=[
            pl.BlockSpec((blk, HIDDEN), lambda i: (i, 0)),
            pl.BlockSpec((blk, HIDDEN), lambda i: (i, 0)),
            pl.BlockSpec((HIDDEN, HIDDEN), lambda i: (0, 0)),
        ],
        out_specs=pl.BlockSpec((blk, HIDDEN), lambda i: (i, 0)),
        out_shape=jax.ShapeDtypeStruct((n, HIDDEN), jnp.float32),
    )(parts[0], parts[1], W_out)

    return hidden_new + jnp.zeros((), dtype=hidden_new.dtype) * n_edge


# speedup vs baseline: 1.9399x; 1.8883x over previous
import jax
import jax.numpy as jnp
from jax import lax
from jax.experimental import pallas as pl
from jax.experimental.pallas import tpu as pltpu
from jax.experimental.pallas import tpu_sc as plsc

N_EDGE = 10000
N_LINE = 320000
HIDDEN = 128
ATTN = 64

NC = 2
NS = 16
NW = NC * NS
LINES_PER_TILE = N_LINE // NW
CHUNK = 40
NCHUNK = LINES_PER_TILE // CHUNK
IDXBLK = 50
NBLKS = NCHUNK // IDXBLK
N_PAD = 10240
ROWS_PER_TILE = N_PAD // NS


def _proj_body(eh_ref, qr_ref, ws_ref, wd_ref, wq_ref, bq_ref, wm_ref,
               p_ref, q_ref, m_ref):
    eh = eh_ref[...]
    qr = qr_ref[...]
    a = jnp.dot(eh, ws_ref[...], preferred_element_type=jnp.float32)
    bc = (jnp.dot(eh, wd_ref[...], preferred_element_type=jnp.float32)
          + jnp.dot(qr, wq_ref[...], preferred_element_type=jnp.float32)
          + bq_ref[...])
    p_ref[...] = jnp.concatenate([a, bc], axis=1)
    q_ref[...] = jnp.concatenate([bc, a], axis=1)
    m_ref[...] = jnp.dot(eh, wm_ref[...], preferred_element_type=jnp.float32)


def _out_body(p0_ref, p1_ref, wo_ref, o_ref):
    o_ref[...] = jnp.dot(p0_ref[...] + p1_ref[...], wo_ref[...],
                         preferred_element_type=jnp.float32)


def _sc_body(u_hbm, q_hbm, src4_hbm, dst2_hbm, wa_hbm, ba_hbm,
             out_hbm, agg_sh, sidx_v, didx2_v, uv, qv, omsg_v, wa_v, ba_v,
             sem_g):
    cid = lax.axis_index("c")
    sid = lax.axis_index("s")
    wid = cid * NS + sid

    def zrow(r, carry):
        for k in range(HIDDEN // 16):
            uv[0, r, pl.ds(k * 16, 16)] = jnp.zeros((16,), jnp.float32)
        return carry
    lax.fori_loop(0, CHUNK, zrow, 0)
    for j in range(ROWS_PER_TILE // CHUNK):
        pltpu.sync_copy(uv.at[0], agg_sh.at[pl.ds(sid * ROWS_PER_TILE + j * CHUNK, CHUNK)])

    pltpu.sync_copy(wa_hbm, wa_v)
    pltpu.sync_copy(ba_hbm, ba_v)
    plsc.subcore_barrier()

    H = CHUNK // 2

    def issue_gathers(r, par):
        tp = lax.shift_right_logical(r, 1)
        pltpu.async_copy(u_hbm.at[sidx_v.at[r, pl.ds(0, H)]],
                         uv.at[par, pl.ds(0, H)], sem_g)
        pltpu.async_copy(u_hbm.at[sidx_v.at[r, pl.ds(H, H)]],
                         uv.at[par, pl.ds(H, H)], sem_g)
        pltpu.async_copy(q_hbm.at[didx2_v.at[tp, pl.ds(par * CHUNK, H)]],
                         qv.at[par, pl.ds(0, H)], sem_g)
        pltpu.async_copy(q_hbm.at[didx2_v.at[tp, pl.ds(par * CHUNK + H, H)]],
                         qv.at[par, pl.ds(H, H)], sem_g)

    def wait_gathers(r, par):
        tp = lax.shift_right_logical(r, 1)
        pltpu.make_async_copy(u_hbm.at[sidx_v.at[r, pl.ds(0, H)]],
                              uv.at[par, pl.ds(0, H)], sem_g).wait()
        pltpu.make_async_copy(u_hbm.at[sidx_v.at[r, pl.ds(H, H)]],
                              uv.at[par, pl.ds(H, H)], sem_g).wait()
        pltpu.make_async_copy(q_hbm.at[didx2_v.at[tp, pl.ds(par * CHUNK, H)]],
                              qv.at[par, pl.ds(0, H)], sem_g).wait()
        pltpu.make_async_copy(q_hbm.at[didx2_v.at[tp, pl.ds(par * CHUNK + H, H)]],
                              qv.at[par, pl.ds(H, H)], sem_g).wait()

    def chunk_step(r, par):
        wait_gathers(r, par)

        @pl.when(r < IDXBLK - 1)
        def _():
            issue_gathers(r + 1, 1 - par)

        wa = [wa_v[pl.ds(k * 16, 16)] for k in range(ATTN // 16)]
        bvec = ba_v[...]

        def line_body(g, c2):
            for u in range(8):
                i = g * 8 + u
                p = []
                for k in range(ATTN // 16):
                    pre = uv[par, i, pl.ds(k * 16, 16)] + qv[par, i, pl.ds(k * 16, 16)]
                    p.append(jnp.maximum(pre, 0.0) * wa[k])
                s = jnp.sum((p[0] + p[1]) + (p[2] + p[3]))
                z = s + bvec
                alpha = 1.0 / (1.0 + jnp.exp(-z))
                for k in range(4):
                    w = uv[par, i, pl.ds(ATTN + k * 16, 16)]
                    lo, hi = plsc.unpack(plsc.bitcast(w, jnp.bfloat16),
                                         format=plsc.PackFormat.INTERLEAVED,
                                         preferred_element_type=jnp.float32)
                    omsg_v[par * CHUNK + i, pl.ds(k * 32, 16)] = lo * alpha
                    omsg_v[par * CHUNK + i, pl.ds(k * 32 + 16, 16)] = hi * alpha
            return c2
        lax.fori_loop(0, CHUNK // 8, line_body, 0)

    for blk in range(NBLKS):
        pltpu.sync_copy(src4_hbm.at[wid, blk], sidx_v)
        pltpu.sync_copy(dst2_hbm.at[wid, blk], didx2_v)
        issue_gathers(0, 0)

        def pair_body(tp, carry):
            chunk_step(2 * tp, 0)
            chunk_step(2 * tp + 1, 1)
            pltpu.sync_copy(omsg_v, agg_sh.at[didx2_v.at[tp]], add=True)
            return carry
        lax.fori_loop(0, IDXBLK // 2, pair_body, 0)

    plsc.subcore_barrier()
    pltpu.sync_copy(agg_sh.at[pl.ds(sid * ROWS_PER_TILE, ROWS_PER_TILE)],
                    out_hbm.at[cid, pl.ds(sid * ROWS_PER_TILE, ROWS_PER_TILE)])


def kernel(edge_hidden, q_rel_emb, line_src, line_dst, n_edge, W_src, W_dst,
           W_qr, b_qr, w_alpha, b_alpha, W_msg, W_out):
    n = edge_hidden.shape[0]
    blk = 1000
    grid = n // blk

    p_proj, q_proj, m_proj = pl.pallas_call(
        _proj_body,
        grid=(grid,),
        in_specs=[
            pl.BlockSpec((blk, HIDDEN), lambda i: (i, 0)),
            pl.BlockSpec((blk, HIDDEN), lambda i: (i, 0)),
            pl.BlockSpec((HIDDEN, ATTN), lambda i: (0, 0)),
            pl.BlockSpec((HIDDEN, ATTN), lambda i: (0, 0)),
            pl.BlockSpec((HIDDEN, ATTN), lambda i: (0, 0)),
            pl.BlockSpec((1, ATTN), lambda i: (0, 0)),
            pl.BlockSpec((HIDDEN, HIDDEN), lambda i: (0, 0)),
        ],
        out_specs=[
            pl.BlockSpec((blk, 2 * ATTN), lambda i: (i, 0)),
            pl.BlockSpec((blk, 2 * ATTN), lambda i: (i, 0)),
            pl.BlockSpec((blk, HIDDEN), lambda i: (i, 0)),
        ],
        out_shape=[
            jax.ShapeDtypeStruct((n, 2 * ATTN), jnp.float32),
            jax.ShapeDtypeStruct((n, 2 * ATTN), jnp.float32),
            jax.ShapeDtypeStruct((n, HIDDEN), jnp.float32),
        ],
    )(edge_hidden, q_rel_emb, W_src, W_dst, W_qr, b_qr.reshape(1, ATTN), W_msg)

    m16 = m_proj.astype(jnp.bfloat16).reshape(n, 4, 2, 16)
    mpk = jnp.stack([m16[:, :, 0, :], m16[:, :, 1, :]], axis=-1)
    mpk = lax.bitcast_convert_type(mpk, jnp.float32).reshape(n, ATTN)
    u_tab = jnp.concatenate([p_proj[:, :ATTN], mpk], axis=1)

    wa_flat = w_alpha.reshape(ATTN)
    ba_vec = jnp.broadcast_to(b_alpha.reshape(1), (16,))
    src32 = line_src.astype(jnp.int32).reshape(NW, NBLKS, IDXBLK, CHUNK)
    dst32 = line_dst.astype(jnp.int32).reshape(NW, NBLKS, IDXBLK // 2, 2 * CHUNK)

    parts = pl.kernel(
        _sc_body,
        out_type=jax.ShapeDtypeStruct((NC, N_PAD, HIDDEN), jnp.float32),
        mesh=plsc.VectorSubcoreMesh(core_axis_name="c", subcore_axis_name="s",
                                    num_cores=NC, num_subcores=NS),
        compiler_params=pltpu.CompilerParams(needs_layout_passes=False),
        scratch_types=[
            pltpu.VMEM_SHARED((N_PAD, HIDDEN), jnp.float32),
            pltpu.VMEM((IDXBLK, CHUNK), jnp.int32),
            pltpu.VMEM((IDXBLK // 2, 2 * CHUNK), jnp.int32),
            pltpu.VMEM((2, CHUNK, HIDDEN), jnp.float32),
            pltpu.VMEM((2, CHUNK, HIDDEN), jnp.float32),
            pltpu.VMEM((2 * CHUNK, HIDDEN), jnp.float32),
            pltpu.VMEM((ATTN,), jnp.float32),
            pltpu.VMEM((16,), jnp.float32),
            pltpu.SemaphoreType.DMA,
        ],
    )(u_tab, q_proj, src32, dst32, wa_flat, ba_vec)

    hidden_new = pl.pallas_call(
        _out_body,
        grid=(grid,),
        in_specs=[
            pl.BlockSpec((blk, HIDDEN), lambda i: (i, 0)),
            pl.BlockSpec((blk, HIDDEN), lambda i: (i, 0)),
            pl.BlockSpec((HIDDEN, HIDDEN), lambda i: (0, 0)),
        ],
        out_specs=pl.BlockSpec((blk, HIDDEN), lambda i: (i, 0)),
        out_shape=jax.ShapeDtypeStruct((n, HIDDEN), jnp.float32),
    )(parts[0], parts[1], W_out)

    return hidden_new + jnp.zeros((), dtype=hidden_new.dtype) * n_edge
